# baseline (speedup 1.0000x reference)
import jax
import jax.numpy as jnp
from jax import lax
from jax.experimental import pallas as pl
from jax.experimental.pallas import tpu as pltpu
from jax.experimental.pallas import tpu_sc as plsc

W, H, N = 1024, 1024, 64
EX, EY, EW, EH = -180.0, -90.0, 360.0, 180.0
PX, PY = 2, 2
KX, KY = 2 * PX + 1, 2 * PY + 1
B = 16384

NC, NS = 2, 16
NWORK = NC * NS
EPW = B // NWORK
NBT = EPW // 128
CR = KY * 128


def _body(x_hbm, y_hbm, tab_hbm, out_hbm, xs_v, ys_v, gx_v, gy_v, src_v,
          rows_v, stage_v, gsem, wsem):
    wid = lax.axis_index("s") * NC + lax.axis_index("c")
    ebase = wid * EPW
    pltpu.sync_copy(x_hbm.at[pl.ds(ebase, EPW)], xs_v)
    pltpu.sync_copy(y_hbm.at[pl.ds(ebase, EPW)], ys_v)
    lanes16 = lax.iota(jnp.int32, 16)

    def bt_body(bt, carry):
        for g in range(8):
            e0 = bt * 128 + g * 16
            xv = xs_v[pl.ds(e0, 16)]
            yv = ys_v[pl.ds(e0, 16)]
            ixv = ((xv - EX) * (W / EW)).astype(jnp.int32)
            iyv = ((yv - EY) * (H / EH)).astype(jnp.int32)
            ixv = jnp.minimum(jnp.maximum(ixv, 0), W - 1)
            iyv = jnp.minimum(jnp.maximum(iyv, 0), H - 1)
            for o in range(-PX, PX + 1):
                gx = ixv + o
                gx = jnp.where(gx < 0, gx + W, gx)
                gx = jnp.where(gx >= W, gx - W, gx)
                gx_v[pl.ds((o + PX) * 128 + g * 16, 16)] = gx * H
            for o in range(-PY, PY + 1):
                gy = iyv + o
                gy = jnp.where(gy < 0, gy + H, gy)
                gy = jnp.where(gy >= H, gy - H, gy)
                gy_v[pl.ds((o + PY) * 128 + g * 16, 16)] = gy
        btg = wid * NBT + bt
        for i in range(KX):
            for j in range(KY):
                for g in range(8):
                    src_v[pl.ds(j * 128 + g * 16, 16)] = (
                        gx_v[pl.ds(i * 128 + g * 16, 16)]
                        + gy_v[pl.ds(j * 128 + g * 16, 16)])
            pltpu.async_copy(tab_hbm.at[src_v], rows_v, gsem).wait()
            rowvs = [lanes16 + (j * 128 + bg * 16)
                     for j in range(KY) for bg in range(8)]

            def n_body(n, c2):
                colv = jnp.full((16,), n, jnp.int32)
                nt = lax.shift_right_logical(n, 3)
                n8 = lax.bitwise_and(n, 7)
                for j in range(KY):
                    for bg in range(8):
                        val = plsc.load_gather(rows_v, [rowvs[j * 8 + bg], colv])
                        stage_v[j * 8 + nt, n8, pl.ds(bg * 16, 16)] = val
                return c2

            lax.fori_loop(0, N, n_body, 0)
            cps = [pltpu.async_copy(stage_v.at[j * 8 + nt],
                                    out_hbm.at[i, j, nt, btg], wsem)
                   for j in range(KY) for nt in range(8)]
            for cp in cps:
                cp.wait()
        return carry

    lax.fori_loop(0, NBT, bt_body, 0)


@jax.jit
def kernel(input, weight):
    xs = input[:, 0]
    ys = input[:, 1]
    tab = weight.reshape(W * H, N)
    mesh = plsc.VectorSubcoreMesh(core_axis_name="c", subcore_axis_name="s")
    out = pl.kernel(
        _body,
        mesh=mesh,
        out_type=jax.ShapeDtypeStruct((KX, KY, N // 8, B // 128, 8, 128),
                                      jnp.float32),
        scratch_types=[
            pltpu.VMEM((EPW,), jnp.float32),
            pltpu.VMEM((EPW,), jnp.float32),
            pltpu.VMEM((KX * 128,), jnp.int32),
            pltpu.VMEM((KY * 128,), jnp.int32),
            pltpu.VMEM((CR,), jnp.int32),
            pltpu.VMEM((CR, N), jnp.float32),
            pltpu.VMEM((KY * 8, 8, 128), jnp.float32),
            pltpu.SemaphoreType.DMA,
            pltpu.SemaphoreType.DMA,
        ],
        compiler_params=pltpu.CompilerParams(use_tc_tiling_on_sc=False,
                                             needs_layout_passes=False),
    )(xs, ys, tab)
    return out.transpose(3, 5, 0, 1, 2, 4).reshape(B, KX, KY, N)


# speedup vs baseline: 1.4873x; 1.4873x over previous
import jax
import jax.numpy as jnp
from jax import lax
from jax.experimental import pallas as pl
from jax.experimental.pallas import tpu as pltpu
from jax.experimental.pallas import tpu_sc as plsc

W, H, N = 1024, 1024, 64
EX, EY, EW, EH = -180.0, -90.0, 360.0, 180.0
PX, PY = 2, 2
KX, KY = 2 * PX + 1, 2 * PY + 1
B = 16384

NC, NS = 2, 16
NWORK = NC * NS
EPW = B // NWORK
NBT = EPW // 128
CR = KY * 128


def _body(x_hbm, y_hbm, tab_hbm, out_hbm, xs_v, ys_v, gx_v, gy_v, src_v,
          rows_v, stage_v, gsem, wsem):
    wid = lax.axis_index("s") * NC + lax.axis_index("c")
    ebase = wid * EPW
    pltpu.sync_copy(x_hbm.at[pl.ds(ebase, EPW)], xs_v)
    pltpu.sync_copy(y_hbm.at[pl.ds(ebase, EPW)], ys_v)
    lanes16 = lax.iota(jnp.int32, 16)

    def bt_body(bt, carry):
        for g in range(8):
            e0 = bt * 128 + g * 16
            xv = xs_v[pl.ds(e0, 16)]
            yv = ys_v[pl.ds(e0, 16)]
            ixv = ((xv - EX) * (W / EW)).astype(jnp.int32)
            iyv = ((yv - EY) * (H / EH)).astype(jnp.int32)
            ixv = jnp.minimum(jnp.maximum(ixv, 0), W - 1)
            iyv = jnp.minimum(jnp.maximum(iyv, 0), H - 1)
            for o in range(-PX, PX + 1):
                gx = ixv + o
                gx = jnp.where(gx < 0, gx + W, gx)
                gx = jnp.where(gx >= W, gx - W, gx)
                gx_v[pl.ds((o + PX) * 128 + g * 16, 16)] = gx * H
            for o in range(-PY, PY + 1):
                gy = iyv + o
                gy = jnp.where(gy < 0, gy + H, gy)
                gy = jnp.where(gy >= H, gy - H, gy)
                gy_v[pl.ds((o + PY) * 128 + g * 16, 16)] = gy
        btg = wid * NBT + bt
        for i in range(KX):
            for j in range(KY):
                for g in range(8):
                    src_v[pl.ds(j * 128 + g * 16, 16)] = (
                        gx_v[pl.ds(i * 128 + g * 16, 16)]
                        + gy_v[pl.ds(j * 128 + g * 16, 16)])
            pltpu.async_copy(tab_hbm.at[src_v], rows_v, gsem).wait()
            n8v = lax.bitwise_and(lanes16, 7)
            hi8 = lax.shift_right_logical(lanes16, 3)
            blkvs = [[jnp.full((16,), j * 8 + nb * 2, jnp.int32) + hi8
                      for nb in range(4)] for j in range(KY)]

            def bl_body(bl, c2):
                colv = jnp.full((16,), bl, jnp.int32)
                for j in range(KY):
                    for nb in range(4):
                        val = rows_v[j * 128 + bl, pl.ds(nb * 16, 16)]
                        plsc.store_scatter(stage_v,
                                           [blkvs[j][nb], n8v, colv], val)
                return c2

            lax.fori_loop(0, 128, bl_body, 0)
            cps = [pltpu.async_copy(
                       stage_v.at[j * 8 + nt, pl.ds(0, 8), pl.ds(0, 128)],
                       out_hbm.at[i, j, nt, btg], wsem)
                   for j in range(KY) for nt in range(8)]
            for cp in cps:
                cp.wait()
        return carry

    lax.fori_loop(0, NBT, bt_body, 0)


@jax.jit
def kernel(input, weight):
    xs = input[:, 0]
    ys = input[:, 1]
    tab = weight.reshape(W * H, N)
    mesh = plsc.VectorSubcoreMesh(core_axis_name="c", subcore_axis_name="s")
    out = pl.kernel(
        _body,
        mesh=mesh,
        out_type=jax.ShapeDtypeStruct((KX, KY, N // 8, B // 128, 8, 128),
                                      jnp.float32),
        scratch_types=[
            pltpu.VMEM((EPW,), jnp.float32),
            pltpu.VMEM((EPW,), jnp.float32),
            pltpu.VMEM((KX * 128,), jnp.int32),
            pltpu.VMEM((KY * 128,), jnp.int32),
            pltpu.VMEM((CR,), jnp.int32),
            pltpu.VMEM((CR, N), jnp.float32),
            pltpu.VMEM((KY * 8, 8, 129), jnp.float32),
            pltpu.SemaphoreType.DMA,
            pltpu.SemaphoreType.DMA,
        ],
        compiler_params=pltpu.CompilerParams(use_tc_tiling_on_sc=False,
                                             needs_layout_passes=False),
    )(xs, ys, tab)
    return out.transpose(3, 5, 0, 1, 2, 4).reshape(B, KX, KY, N)


# speedup vs baseline: 2.1459x; 1.4428x over previous
import jax
import jax.numpy as jnp
from jax import lax
from jax.experimental import pallas as pl
from jax.experimental.pallas import tpu as pltpu
from jax.experimental.pallas import tpu_sc as plsc

W, H, N = 1024, 1024, 64
EX, EY, EW, EH = -180.0, -90.0, 360.0, 180.0
PX, PY = 2, 2
KX, KY = 2 * PX + 1, 2 * PY + 1
B = 16384

NC, NS = 2, 16
NWORK = NC * NS
EPW = B // NWORK
NBT = EPW // 128
CR = KY * 128
HW = W // 2

_XB = 8


def _tc_pair_body(wt0_ref, wt1_ref, out_ref):
    t0 = jnp.transpose(wt0_ref[...], (0, 2, 1))
    t1 = jnp.transpose(wt1_ref[...], (0, 2, 1))
    out_ref[...] = jnp.concatenate([t0, t1], axis=-1)


def _linearize_table(weight):
    wt = weight.transpose(0, 2, 1)
    lin = pl.pallas_call(
        _tc_pair_body,
        grid=(HW // _XB,),
        in_specs=[
            pl.BlockSpec((_XB, N, H), lambda i: (i, 0, 0)),
            pl.BlockSpec((_XB, N, H), lambda i: (i + HW // _XB, 0, 0)),
        ],
        out_specs=pl.BlockSpec((_XB, H, 2 * N), lambda i: (i, 0, 0)),
        out_shape=jax.ShapeDtypeStruct((HW, H, 2 * N), jnp.float32),
    )(wt, wt)
    return lin.reshape(HW * H, 2 * N)


def _sc_body(x_hbm, y_hbm, tab_hbm, out_hbm, xs_v, ys_v, gq_v, gy_v, h_v,
             src_v, rows_v, stage_v, gsem, wsem):
    wid = lax.axis_index("s") * NC + lax.axis_index("c")
    ebase = wid * EPW
    pltpu.sync_copy(x_hbm.at[pl.ds(ebase, EPW)], xs_v)
    pltpu.sync_copy(y_hbm.at[pl.ds(ebase, EPW)], ys_v)
    lanes16 = lax.iota(jnp.int32, 16)
    n8v = lax.bitwise_and(lanes16, 7)
    hi8 = lax.shift_right_logical(lanes16, 3)
    blkvs = [[jnp.full((16,), j * 8 + nb * 2, jnp.int32) + hi8
              for nb in range(4)] for j in range(KY)]

    def bt_body(bt, carry):
        for g in range(8):
            e0 = bt * 128 + g * 16
            xv = xs_v[pl.ds(e0, 16)]
            yv = ys_v[pl.ds(e0, 16)]
            ixv = ((xv - EX) * (W / EW)).astype(jnp.int32)
            iyv = ((yv - EY) * (H / EH)).astype(jnp.int32)
            ixv = jnp.minimum(jnp.maximum(ixv, 0), W - 1)
            iyv = jnp.minimum(jnp.maximum(iyv, 0), H - 1)
            for o in range(-PX, PX + 1):
                gx = ixv + o
                gx = jnp.where(gx < 0, gx + W, gx)
                gx = jnp.where(gx >= W, gx - W, gx)
                gq_v[pl.ds((o + PX) * 128 + g * 16, 16)] = (
                    lax.bitwise_and(gx, HW - 1) * H)
                h_v[pl.ds((o + PX) * 128 + g * 16, 16)] = (
                    lax.shift_right_logical(gx, 9))
            for o in range(-PY, PY + 1):
                gy = iyv + o
                gy = jnp.where(gy < 0, gy + H, gy)
                gy = jnp.where(gy >= H, gy - H, gy)
                gy_v[pl.ds((o + PY) * 128 + g * 16, 16)] = gy
        btg = wid * NBT + bt
        for i in range(KX):
            for j in range(KY):
                for g in range(8):
                    src_v[pl.ds(j * 128 + g * 16, 16)] = (
                        gq_v[pl.ds(i * 128 + g * 16, 16)]
                        + gy_v[pl.ds(j * 128 + g * 16, 16)])
            pltpu.async_copy(tab_hbm.at[src_v], rows_v, gsem).wait()
            def bl_body(bl, c2):
                colv = jnp.full((16,), bl, jnp.int32)
                hm = plsc.load_gather(
                    h_v, [jnp.full((16,), i * 128 + bl, jnp.int32)]) > 0
                for j in range(KY):
                    for nb in range(4):
                        lo = rows_v[j * 128 + bl, pl.ds(nb * 16, 16)]
                        hi = rows_v[j * 128 + bl, pl.ds(N + nb * 16, 16)]
                        val = jnp.where(hm, hi, lo)
                        plsc.store_scatter(stage_v,
                                           [blkvs[j][nb], n8v, colv], val)
                return c2

            lax.fori_loop(0, 128, bl_body, 0)
            cps = [pltpu.async_copy(
                       stage_v.at[j * 8 + nt, pl.ds(0, 8), pl.ds(0, 128)],
                       out_hbm.at[i, j, nt, btg], wsem)
                   for j in range(KY) for nt in range(8)]
            for cp in cps:
                cp.wait()
        return carry

    lax.fori_loop(0, NBT, bt_body, 0)


@jax.jit
def kernel(input, weight):
    xs = input[:, 0]
    ys = input[:, 1]
    tab = _linearize_table(weight)
    mesh = plsc.VectorSubcoreMesh(core_axis_name="c", subcore_axis_name="s")
    out = pl.kernel(
        _sc_body,
        mesh=mesh,
        out_type=jax.ShapeDtypeStruct((KX, KY, N // 8, B // 128, 8, 128),
                                      jnp.float32),
        scratch_types=[
            pltpu.VMEM((EPW,), jnp.float32),
            pltpu.VMEM((EPW,), jnp.float32),
            pltpu.VMEM((KX * 128,), jnp.int32),
            pltpu.VMEM((KY * 128,), jnp.int32),
            pltpu.VMEM((KX * 128,), jnp.int32),
            pltpu.VMEM((CR,), jnp.int32),
            pltpu.VMEM((CR, 2 * N), jnp.float32),
            pltpu.VMEM((KY * 8, 8, 129), jnp.float32),
            pltpu.SemaphoreType.DMA,
            pltpu.SemaphoreType.DMA,
        ],
        compiler_params=pltpu.CompilerParams(use_tc_tiling_on_sc=False,
                                             needs_layout_passes=False),
    )(xs, ys, tab)
    return out.transpose(3, 5, 0, 1, 2, 4).reshape(B, KX, KY, N)


# speedup vs baseline: 2.4020x; 1.1193x over previous
import jax
import jax.numpy as jnp
from jax import lax
from jax.experimental import pallas as pl
from jax.experimental.pallas import tpu as pltpu
from jax.experimental.pallas import tpu_sc as plsc

W, H, N = 1024, 1024, 64
EX, EY, EW, EH = -180.0, -90.0, 360.0, 180.0
PX, PY = 2, 2
KX, KY = 2 * PX + 1, 2 * PY + 1
B = 16384

NC, NS = 2, 16
NWORK = NC * NS
EPW = B // NWORK
NBT = EPW // 128
CR = KY * 128
HW = W // 2

_XB = 8


def _tc_pair_body(wt0_ref, wt1_ref, out_ref):
    eye = jnp.eye(N, dtype=jnp.float32)
    for x in range(_XB):
        t0 = lax.dot_general(wt0_ref[x], eye, (((0,), (0,)), ((), ())))
        t1 = lax.dot_general(wt1_ref[x], eye, (((0,), (0,)), ((), ())))
        out_ref[x] = jnp.concatenate([t0, t1], axis=-1)


def _linearize_table(weight):
    wt = weight.transpose(0, 2, 1)
    lin = pl.pallas_call(
        _tc_pair_body,
        grid=(HW // _XB,),
        in_specs=[
            pl.BlockSpec((_XB, N, H), lambda i: (i, 0, 0)),
            pl.BlockSpec((_XB, N, H), lambda i: (i + HW // _XB, 0, 0)),
        ],
        out_specs=pl.BlockSpec((_XB, H, 2 * N), lambda i: (i, 0, 0)),
        out_shape=jax.ShapeDtypeStruct((HW, H, 2 * N), jnp.float32),
    )(wt, wt)
    return lin.reshape(HW * H, 2 * N)


def _sc_body(x_hbm, y_hbm, tab_hbm, out_hbm, xs_v, ys_v, gq_v, gy_v, h_v,
             src_v, src2_v, rows_v, rows2_v, stage_v, gsem, gsem2, wsem):
    wid = lax.axis_index("s") * NC + lax.axis_index("c")
    ebase = wid * EPW
    pltpu.sync_copy(x_hbm.at[pl.ds(ebase, EPW)], xs_v)
    pltpu.sync_copy(y_hbm.at[pl.ds(ebase, EPW)], ys_v)
    lanes16 = lax.iota(jnp.int32, 16)
    n8v = lax.bitwise_and(lanes16, 7)
    hi8 = lax.shift_right_logical(lanes16, 3)
    blkvs = [[jnp.full((16,), j * 8 + nb * 2, jnp.int32) + hi8
              for nb in range(4)] for j in range(KY)]

    def bt_body(bt, carry):
        for g in range(8):
            e0 = bt * 128 + g * 16
            xv = xs_v[pl.ds(e0, 16)]
            yv = ys_v[pl.ds(e0, 16)]
            ixv = ((xv - EX) * (W / EW)).astype(jnp.int32)
            iyv = ((yv - EY) * (H / EH)).astype(jnp.int32)
            ixv = jnp.minimum(jnp.maximum(ixv, 0), W - 1)
            iyv = jnp.minimum(jnp.maximum(iyv, 0), H - 1)
            for o in range(-PX, PX + 1):
                gx = ixv + o
                gx = jnp.where(gx < 0, gx + W, gx)
                gx = jnp.where(gx >= W, gx - W, gx)
                gq_v[pl.ds((o + PX) * 128 + g * 16, 16)] = (
                    lax.bitwise_and(gx, HW - 1) * H)
                h_v[pl.ds((o + PX) * 128 + g * 16, 16)] = (
                    lax.shift_right_logical(gx, 9))
            for o in range(-PY, PY + 1):
                gy = iyv + o
                gy = jnp.where(gy < 0, gy + H, gy)
                gy = jnp.where(gy >= H, gy - H, gy)
                gy_v[pl.ds((o + PY) * 128 + g * 16, 16)] = gy
        btg = wid * NBT + bt
        srcs, rows, sems = (src_v, src2_v), (rows_v, rows2_v), (gsem, gsem2)

        def build_and_fire(p):
            i, h2 = divmod(p, 2)
            sv, rv, sem = srcs[p % 2], rows[p % 2], sems[p % 2]
            for j in range(KY):
                for g in range(4):
                    sv[pl.ds(j * 64 + g * 16, 16)] = (
                        gq_v[pl.ds(i * 128 + h2 * 64 + g * 16, 16)]
                        + gy_v[pl.ds(j * 128 + h2 * 64 + g * 16, 16)])
            return pltpu.async_copy(tab_hbm.at[sv], rv, sem)

        cp = build_and_fire(0)
        for p in range(2 * KX):
            nxt = build_and_fire(p + 1) if p + 1 < 2 * KX else None
            cp.wait()
            i, h2 = divmod(p, 2)
            rv = rows[p % 2]

            def bl_body(bl, c2):
                colv = jnp.full((16,), bl, jnp.int32) + (h2 * 64)
                hm = plsc.load_gather(
                    h_v,
                    [jnp.full((16,), i * 128 + h2 * 64, jnp.int32) + bl]) > 0
                for j in range(KY):
                    for nb in range(4):
                        lo = rv[j * 64 + bl, pl.ds(nb * 16, 16)]
                        hi = rv[j * 64 + bl, pl.ds(N + nb * 16, 16)]
                        val = jnp.where(hm, hi, lo)
                        plsc.store_scatter(stage_v,
                                           [blkvs[j][nb], n8v, colv], val)
                return c2

            lax.fori_loop(0, 64, bl_body, 0)
            if p % 2 == 1:
                cps = [pltpu.async_copy(
                           stage_v.at[j * 8 + nt, pl.ds(0, 8), pl.ds(0, 128)],
                           out_hbm.at[i, j, nt, btg], wsem)
                       for j in range(KY) for nt in range(8)]
                for c in cps:
                    c.wait()
            cp = nxt
        return carry

    lax.fori_loop(0, NBT, bt_body, 0)


@jax.jit
def kernel(input, weight):
    xs = input[:, 0]
    ys = input[:, 1]
    tab = _linearize_table(weight)
    mesh = plsc.VectorSubcoreMesh(core_axis_name="c", subcore_axis_name="s")
    out = pl.kernel(
        _sc_body,
        mesh=mesh,
        out_type=jax.ShapeDtypeStruct((KX, KY, N // 8, B // 128, 8, 128),
                                      jnp.float32),
        scratch_types=[
            pltpu.VMEM((EPW,), jnp.float32),
            pltpu.VMEM((EPW,), jnp.float32),
            pltpu.VMEM((KX * 128,), jnp.int32),
            pltpu.VMEM((KY * 128,), jnp.int32),
            pltpu.VMEM((KX * 128,), jnp.int32),
            pltpu.VMEM((CR // 2,), jnp.int32),
            pltpu.VMEM((CR // 2,), jnp.int32),
            pltpu.VMEM((CR // 2, 2 * N), jnp.float32),
            pltpu.VMEM((CR // 2, 2 * N), jnp.float32),
            pltpu.VMEM((KY * 8, 8, 129), jnp.float32),
            pltpu.SemaphoreType.DMA,
            pltpu.SemaphoreType.DMA,
            pltpu.SemaphoreType.DMA,
        ],
        compiler_params=pltpu.CompilerParams(use_tc_tiling_on_sc=False,
                                             needs_layout_passes=False),
    )(xs, ys, tab)
    return out.transpose(3, 5, 0, 1, 2, 4).reshape(B, KX, KY, N)


# speedup vs baseline: 2.4051x; 1.0013x over previous
import jax
import jax.numpy as jnp
from jax import lax
from jax.experimental import pallas as pl
from jax.experimental.pallas import tpu as pltpu
from jax.experimental.pallas import tpu_sc as plsc

W, H, N = 1024, 1024, 64
EX, EY, EW, EH = -180.0, -90.0, 360.0, 180.0
PX, PY = 2, 2
KX, KY = 2 * PX + 1, 2 * PY + 1
B = 16384

NC, NS = 2, 16
NWORK = NC * NS
EPW = B // NWORK
NBT = EPW // 128
CR = KY * 128
HW = W // 2

_XB = 8


def _tc_pair_body(wt0_ref, wt1_ref, out_ref):
    t0 = jnp.transpose(wt0_ref[...], (0, 2, 1))
    t1 = jnp.transpose(wt1_ref[...], (0, 2, 1))
    out_ref[...] = jnp.concatenate([t0, t1], axis=-1)


def _linearize_table(weight):
    wt = weight.transpose(0, 2, 1)
    lin = pl.pallas_call(
        _tc_pair_body,
        grid=(HW // _XB,),
        in_specs=[
            pl.BlockSpec((_XB, N, H), lambda i: (i, 0, 0)),
            pl.BlockSpec((_XB, N, H), lambda i: (i + HW // _XB, 0, 0)),
        ],
        out_specs=pl.BlockSpec((_XB, H, 2 * N), lambda i: (i, 0, 0)),
        out_shape=jax.ShapeDtypeStruct((HW, H, 2 * N), jnp.float32),
    )(wt, wt)
    return lin.reshape(HW * H, 2 * N)


def _sc_body(x_hbm, y_hbm, tab_hbm, out_hbm, xs_v, ys_v, gq_v, gy_v, h_v,
             src_v, src2_v, rows_v, rows2_v, stage_v, gsem, gsem2, wsem):
    wid = lax.axis_index("s") * NC + lax.axis_index("c")
    ebase = wid * EPW
    pltpu.sync_copy(x_hbm.at[pl.ds(ebase, EPW)], xs_v)
    pltpu.sync_copy(y_hbm.at[pl.ds(ebase, EPW)], ys_v)
    lanes16 = lax.iota(jnp.int32, 16)
    n8v = lax.bitwise_and(lanes16, 7)
    hi8 = lax.shift_right_logical(lanes16, 3)
    blkvs = [[jnp.full((16,), j * 8 + nb * 2, jnp.int32) + hi8
              for nb in range(4)] for j in range(KY)]

    def bt_body(bt, carry):
        for g in range(8):
            e0 = bt * 128 + g * 16
            xv = xs_v[pl.ds(e0, 16)]
            yv = ys_v[pl.ds(e0, 16)]
            ixv = ((xv - EX) * (W / EW)).astype(jnp.int32)
            iyv = ((yv - EY) * (H / EH)).astype(jnp.int32)
            ixv = jnp.minimum(jnp.maximum(ixv, 0), W - 1)
            iyv = jnp.minimum(jnp.maximum(iyv, 0), H - 1)
            for o in range(-PX, PX + 1):
                gx = ixv + o
                gx = jnp.where(gx < 0, gx + W, gx)
                gx = jnp.where(gx >= W, gx - W, gx)
                gq_v[pl.ds((o + PX) * 128 + g * 16, 16)] = (
                    lax.bitwise_and(gx, HW - 1) * H)
                h_v[pl.ds((o + PX) * 128 + g * 16, 16)] = (
                    lax.shift_right_logical(gx, 9))
            for o in range(-PY, PY + 1):
                gy = iyv + o
                gy = jnp.where(gy < 0, gy + H, gy)
                gy = jnp.where(gy >= H, gy - H, gy)
                gy_v[pl.ds((o + PY) * 128 + g * 16, 16)] = gy
        btg = wid * NBT + bt
        srcs, rows, sems = (src_v, src2_v), (rows_v, rows2_v), (gsem, gsem2)

        def build_and_fire(p):
            i, h2 = divmod(p, 2)
            sv, rv, sem = srcs[p % 2], rows[p % 2], sems[p % 2]
            for j in range(KY):
                for g in range(4):
                    sv[pl.ds(j * 64 + g * 16, 16)] = (
                        gq_v[pl.ds(i * 128 + h2 * 64 + g * 16, 16)]
                        + gy_v[pl.ds(j * 128 + h2 * 64 + g * 16, 16)])
            return pltpu.async_copy(tab_hbm.at[sv], rv, sem)

        cp = build_and_fire(0)
        for p in range(2 * KX):
            nxt = build_and_fire(p + 1) if p + 1 < 2 * KX else None
            cp.wait()
            i, h2 = divmod(p, 2)
            rv = rows[p % 2]

            def bl_body(bl, c2):
                colv = jnp.full((16,), bl, jnp.int32) + (h2 * 64)
                hm = plsc.load_gather(
                    h_v,
                    [jnp.full((16,), i * 128 + h2 * 64, jnp.int32) + bl]) > 0
                for j in range(KY):
                    for nb in range(4):
                        lo = rv[j * 64 + bl, pl.ds(nb * 16, 16)]
                        hi = rv[j * 64 + bl, pl.ds(N + nb * 16, 16)]
                        val = jnp.where(hm, hi, lo)
                        plsc.store_scatter(stage_v,
                                           [blkvs[j][nb], n8v, colv], val)
                return c2

            lax.fori_loop(0, 64, bl_body, 0)
            if p % 2 == 1:
                cps = [pltpu.async_copy(
                           stage_v.at[j * 8 + nt, pl.ds(0, 8), pl.ds(0, 128)],
                           out_hbm.at[i, j, nt, btg], wsem)
                       for j in range(KY) for nt in range(8)]
                for c in cps:
                    c.wait()
            cp = nxt
        return carry

    lax.fori_loop(0, NBT, bt_body, 0)


@jax.jit
def kernel(input, weight):
    xs = input[:, 0]
    ys = input[:, 1]
    tab = _linearize_table(weight)
    mesh = plsc.VectorSubcoreMesh(core_axis_name="c", subcore_axis_name="s")
    out = pl.kernel(
        _sc_body,
        mesh=mesh,
        out_type=jax.ShapeDtypeStruct((KX, KY, N // 8, B // 128, 8, 128),
                                      jnp.float32),
        scratch_types=[
            pltpu.VMEM((EPW,), jnp.float32),
            pltpu.VMEM((EPW,), jnp.float32),
            pltpu.VMEM((KX * 128,), jnp.int32),
            pltpu.VMEM((KY * 128,), jnp.int32),
            pltpu.VMEM((KX * 128,), jnp.int32),
            pltpu.VMEM((CR // 2,), jnp.int32),
            pltpu.VMEM((CR // 2,), jnp.int32),
            pltpu.VMEM((CR // 2, 2 * N), jnp.float32),
            pltpu.VMEM((CR // 2, 2 * N), jnp.float32),
            pltpu.VMEM((KY * 8, 8, 129), jnp.float32),
            pltpu.SemaphoreType.DMA,
            pltpu.SemaphoreType.DMA,
            pltpu.SemaphoreType.DMA,
        ],
        compiler_params=pltpu.CompilerParams(use_tc_tiling_on_sc=False,
                                             needs_layout_passes=False),
    )(xs, ys, tab)
    return out.transpose(3, 5, 0, 1, 2, 4).reshape(B, KX, KY, N)
